# baseline (speedup 1.0000x reference)
import functools

import numpy as np
import jax
import jax.numpy as jnp
from jax import lax
from jax.experimental import pallas as pl
from jax.experimental.pallas import tpu as pltpu
from jax.experimental.pallas import tpu_sc as plsc

_NC = 2
_NS = 16
_NW = _NC * _NS
_LANES = 16


def _pos_encoding(dk: int, length: int) -> np.ndarray:
    i = np.arange(dk)
    even = (i % 2 == 0).astype(np.float64)
    power = 10000.0 ** (2.0 * i / dk)
    pos = np.arange(length)[:, None]
    ang = pos / power[None, :]
    return (np.sin(ang) * even + np.cos(ang) * (1.0 - even)).astype(np.float32)


@functools.partial(jax.jit, static_argnames=("interpret",))
def kernel(tokens, table, *, interpret=False):
    B, L = tokens.shape
    V, D = table.shape
    assert B % _NW == 0 and D % _LANES == 0
    seq_per_w = B // _NW
    nch = -(-L // 128)
    assert L % nch == 0
    ch = L // nch

    pos = jnp.asarray(_pos_encoding(D, L))
    tokens_c = tokens.astype(jnp.int32).reshape(B, nch, ch)

    mesh = plsc.VectorSubcoreMesh(
        core_axis_name="c", subcore_axis_name="s",
        num_cores=_NC, num_subcores=_NS,
    )

    @functools.partial(
        pl.kernel,
        out_type=jax.ShapeDtypeStruct((B, L, D), jnp.float32),
        mesh=mesh,
        scratch_types=[
            [pltpu.VMEM((ch,), jnp.int32) for _ in range(nch)],
            pltpu.VMEM((L, D), jnp.float32),
            pltpu.VMEM((L, D), jnp.float32),
            pltpu.SemaphoreType.DMA,
        ],
        compiler_params=pltpu.CompilerParams(use_tc_tiling_on_sc=False),
        interpret=interpret,
    )
    def emb_kernel(tokens_hbm, pos_hbm, table_hbm, out_hbm,
                   idx_vs, rows_v, pos_v, sem):
        wid = lax.axis_index("s") * _NC + lax.axis_index("c")
        pltpu.sync_copy(pos_hbm, pos_v)

        @pl.loop(0, seq_per_w)
        def _seq(j):
            seq = wid * seq_per_w + j
            for h in range(nch):
                pltpu.sync_copy(tokens_hbm.at[seq, h], idx_vs[h])
            descs = [
                pltpu.async_copy(
                    table_hbm.at[idx_vs[h]],
                    rows_v.at[pl.ds(h * ch, ch)],
                    sem,
                )
                for h in range(nch)
            ]
            for d in descs:
                d.wait()

            @pl.loop(0, L)
            def _row(r):
                for c in range(D // _LANES):
                    sl = pl.ds(c * _LANES, _LANES)
                    rows_v[r, sl] = rows_v[r, sl] + pos_v[r, sl]

            pltpu.sync_copy(rows_v, out_hbm.at[seq])

    return emb_kernel(tokens_c, pos, table)


# speedup vs baseline: 1.0810x; 1.0810x over previous
import functools

import numpy as np
import jax
import jax.numpy as jnp
from jax import lax
from jax.experimental import pallas as pl
from jax.experimental.pallas import tpu as pltpu
from jax.experimental.pallas import tpu_sc as plsc

_NC = 2
_NS = 16
_NW = _NC * _NS
_LANES = 16
_NBUF = 4


def _pos_encoding(dk: int, length: int) -> np.ndarray:
    i = np.arange(dk)
    even = (i % 2 == 0).astype(np.float64)
    power = 10000.0 ** (2.0 * i / dk)
    pos = np.arange(length)[:, None]
    ang = pos / power[None, :]
    return (np.sin(ang) * even + np.cos(ang) * (1.0 - even)).astype(np.float32)


@functools.partial(jax.jit, static_argnames=("interpret",))
def kernel(tokens, table, *, interpret=False):
    B, L = tokens.shape
    V, D = table.shape
    assert B % _NW == 0 and D % _LANES == 0
    spw = B // _NW
    assert spw % _NBUF == 0
    nch = -(-L // 128)
    assert L % nch == 0
    ch = L // nch

    pos = jnp.asarray(_pos_encoding(D, L))
    tokens_c = tokens.astype(jnp.int32).reshape(_NW, spw, nch, ch)

    mesh = plsc.VectorSubcoreMesh(
        core_axis_name="c", subcore_axis_name="s",
        num_cores=_NC, num_subcores=_NS,
    )

    @functools.partial(
        pl.kernel,
        out_type=jax.ShapeDtypeStruct((B, L, D), jnp.float32),
        mesh=mesh,
        scratch_types=[
            pltpu.VMEM((spw, nch, ch), jnp.int32),
            pltpu.VMEM((_NBUF, L, D), jnp.float32),
            pltpu.VMEM((L, D), jnp.float32),
            [pltpu.SemaphoreType.DMA for _ in range(_NBUF)],
            [pltpu.SemaphoreType.DMA for _ in range(_NBUF)],
        ],
        compiler_params=pltpu.CompilerParams(use_tc_tiling_on_sc=False),
        interpret=interpret,
    )
    def emb_kernel(tokens_hbm, pos_hbm, table_hbm, out_hbm,
                   idx_all, rows, pos_v, sem_g, sem_o):
        wid = lax.axis_index("s") * _NC + lax.axis_index("c")
        base = wid * spw
        pltpu.sync_copy(pos_hbm, pos_v)
        pltpu.sync_copy(tokens_hbm.at[wid], idx_all)

        def gather(j, b, issue):
            mk = pltpu.async_copy if issue else pltpu.make_async_copy
            return [
                mk(table_hbm.at[idx_all.at[j, h]],
                   rows.at[b, pl.ds(h * ch, ch)],
                   sem_g[b])
                for h in range(nch)
            ]

        for k in range(_NBUF - 1):
            gather(k, k, True)

        @pl.loop(0, spw, step=_NBUF)
        def _outer(jo):
            for b in range(_NBUF):
                j = jo + b
                for d in gather(j, b, False):
                    d.wait()

                @pl.loop(0, L)
                def _row(r):
                    for c in range(D // _LANES):
                        sl = pl.ds(c * _LANES, _LANES)
                        rows[b, r, sl] = rows[b, r, sl] + pos_v[r, sl]

                bp = (b - 1) % _NBUF

                @pl.when(j + _NBUF - 1 < spw)
                def _():
                    gather(j + _NBUF - 1, bp, True)

                pltpu.sync_copy(rows.at[b], out_hbm.at[base + j])

    return emb_kernel(tokens_c, pos, table)
